# baseline (speedup 1.0000x reference)
import functools
import numpy as np
import jax
import jax.numpy as jnp
from jax import lax
from jax.experimental import pallas as pl
from jax.experimental.pallas import tpu as pltpu
from jax.experimental.pallas import tpu_sc as plsc

_NUM_ATOMS = np.array([5, 11, 8, 8, 6, 9, 9, 4, 10, 8, 8, 9, 8, 11, 7, 6, 7, 14, 12, 7],
                      dtype=np.float32)
_SC_ELEMS = ["C", "CCCNCNN", "CCON", "CCOO", "CS", "CCCON", "CCCOO", "", "CCNCCN",
             "CCCC", "CCCC", "CCCCN", "CCSC", "CCCCCCC", "CCC", "CO", "COC",
             "CCCCNCCCCC", "CCCCCCCO", "CCC"]
_VDW = {"C": 1.7, "N": 1.55, "O": 1.52, "S": 1.8}


def _build_vdw_table():
    R = np.zeros((20, 14), dtype=np.float32)
    for i, sc in enumerate(_SC_ELEMS):
        for j, e in enumerate("NCCO" + sc):
            R[i, j] = _VDW[e]
    return R


_VDW_R = _build_vdw_table()

_LG = 16
_KP = 32
_RB = 256
_IW = 128
_CH_E = 512
_EPS4 = 0.001 / 4.0
_NEG = -30.0


def _sc_gather4(tables, gidx, n_edges):
    info = plsc.get_sparse_core_info()
    nc, ns = info.num_cores, info.num_subcores
    nw = nc * ns
    idx_rows = gidx.shape[1]
    e_per_w = idx_rows * _IW
    chunks = e_per_w // _CH_E
    q_per_ch = _CH_E // _IW
    mesh = plsc.VectorSubcoreMesh(core_axis_name="c", subcore_axis_name="s")
    out_t = jax.ShapeDtypeStruct((n_edges, _LG), jnp.float32)
    buf_t = pltpu.VMEM((_CH_E, _LG), jnp.float32)

    @functools.partial(
        pl.kernel,
        mesh=mesh,
        compiler_params=pltpu.CompilerParams(use_tc_tiling_on_sc=False),
        out_type=(out_t,) * 4,
        scratch_types=[
            pltpu.VMEM((idx_rows, _IW), jnp.int32),
            buf_t, buf_t, buf_t, buf_t,
            pltpu.SemaphoreType.DMA,
            pltpu.SemaphoreType.DMA,
            pltpu.SemaphoreType.DMA,
            pltpu.SemaphoreType.DMA,
        ],
    )
    def gather_kernel(t0, t1, t2, t3, gidx_hbm, o0, o1, o2, o3,
                      idx_v, r0, r1, r2, r3, s0, s1, s2, s3):
        wid = lax.axis_index("s") * nc + lax.axis_index("c")
        e_base = wid * e_per_w
        pltpu.sync_copy(gidx_hbm.at[wid], idx_v)
        tabs = ((t0, r0, s0), (t1, r1, s1), (t2, r2, s2), (t3, r3, s3))
        outs = (o0, o1, o2, o3)

        def body(c, carry):
            cps = []
            for (t, r, s) in tabs:
                for q in range(q_per_ch):
                    cps.append(pltpu.async_copy(
                        t.at[idx_v.at[c * q_per_ch + q]],
                        r.at[pl.ds(q * _IW, _IW)], s))
            for cp in cps:
                cp.wait()
            dst = pl.ds(e_base + c * _CH_E, _CH_E)
            for (t, r, s), o in zip(tabs, outs):
                pltpu.sync_copy(r, o.at[dst])
            return carry

        lax.fori_loop(0, chunks, body, 0)

    return gather_kernel(tables[0], tables[1], tables[2], tables[3], gidx)


def _tc_body(gx0_ref, gx1_ref, gx2_ref, gr_ref,
             sx0_ref, sx1_ref, sx2_ref, sr_ref, main_ref, self_ref):
    g0, g1, g2, gr = gx0_ref[...], gx1_ref[...], gx2_ref[...], gr_ref[...]
    s0, s1, s2, sr = sx0_ref[...], sx1_ref[...], sx2_ref[...], sr_ref[...]

    acc = jnp.zeros_like(g0)
    accs = jnp.zeros_like(s0)
    for a in range(4, 14):
        qx0 = s0[:, a:a + 1]
        qx1 = s1[:, a:a + 1]
        qx2 = s2[:, a:a + 1]
        qr = sr[:, a:a + 1]
        d2 = (qx0 - g0) ** 2 + ((qx1 - g1) ** 2 + ((qx2 - g2) ** 2 + _EPS4))
        acc = acc + jnp.tanh((qr + gr) - jnp.sqrt(d2))
        d2s = (qx0 - s0) ** 2 + ((qx1 - s1) ** 2 + ((qx2 - s2) ** 2 + _EPS4))
        accs = accs + jnp.tanh((qr + sr) - jnp.sqrt(d2s))

    main_ref[...] = jnp.sum(acc, axis=1, keepdims=True)
    self_ref[...] = jnp.sum(accs, axis=1, keepdims=True)


def _tc_compute(gx, sq, n_rows, interpret=False):
    rows_b = _RB * (_KP * _LG // 128)
    grid = (n_rows // rows_b,)
    gspec = pl.BlockSpec((rows_b, 128), lambda i: (i, 0))
    sspec = pl.BlockSpec((rows_b, _LG), lambda i: (i, 0))
    ospec = pl.BlockSpec((rows_b, 1), lambda i: (i, 0))
    o_t = jax.ShapeDtypeStruct((n_rows, 1), jnp.float32)
    return pl.pallas_call(
        _tc_body,
        grid=grid,
        in_specs=[gspec] * 4 + [sspec] * 4,
        out_specs=(ospec, ospec),
        out_shape=(o_t, o_t),
        interpret=interpret,
    )(*gx, *sq)


def _build_tables(X, C, S):
    B, N, A, _ = X.shape
    onehot = (S[:, :, None] == jnp.arange(20, dtype=S.dtype)).astype(jnp.float32)
    rmat = jnp.asarray(_VDW_R * 0.25 + 0.0875)
    rrow = jnp.dot(onehot, rmat, precision=jax.lax.Precision.HIGHEST)
    apr = (C > 0).astype(jnp.float32) * jnp.dot(onehot, jnp.asarray(_NUM_ATOMS),
                                                precision=jax.lax.Precision.HIGHEST)
    mask = jnp.arange(A, dtype=jnp.float32).reshape(1, 1, A) < apr[:, :, None]
    r4 = jnp.where(mask, rrow, _NEG)
    pad0 = jnp.zeros((B, N, _LG - A), jnp.float32)
    padn = jnp.full((B, N, _LG - A), _NEG, jnp.float32)
    xh = X * 0.5
    zrow = jnp.zeros((1, _LG), jnp.float32)
    nrow = jnp.full((1, _LG), _NEG, jnp.float32)
    tx0 = jnp.concatenate(
        [jnp.concatenate([xh[:, :, :, 0], pad0], -1).reshape(B * N, _LG), zrow], 0)
    tx1 = jnp.concatenate(
        [jnp.concatenate([xh[:, :, :, 1], pad0], -1).reshape(B * N, _LG), zrow], 0)
    tx2 = jnp.concatenate(
        [jnp.concatenate([xh[:, :, :, 2], pad0], -1).reshape(B * N, _LG), zrow], 0)
    tr = jnp.concatenate(
        [jnp.concatenate([r4, padn], -1).reshape(B * N, _LG), nrow], 0)
    return (tx0, tx1, tx2, tr)


def kernel(X, C, S, edge_idx):
    B, N, A, _ = X.shape
    Kn = edge_idx.shape[2]
    n_edges = B * N * _KP
    rep = _KP * _LG // 128

    tables = _build_tables(X, C, S)

    eidx = edge_idx.astype(jnp.int32)
    gidx = eidx + (jnp.arange(B, dtype=jnp.int32) * N)[:, None, None]
    gidx = jnp.concatenate(
        [gidx, jnp.full((B, N, _KP - Kn), B * N, jnp.int32)], axis=-1).reshape(-1)
    nw = 32
    gidx3 = gidx.reshape(nw, n_edges // (nw * _IW), _IW)

    g = _sc_gather4(tables, gidx3, n_edges)
    gx = tuple(t.reshape(n_edges * _LG // 128, 128) for t in g)
    sq = tuple(jnp.repeat(t[:B * N], rep, axis=0) for t in tables)

    main, selfs = _tc_compute(gx, sq, B * N * rep)

    ms = main.reshape(B, N, rep).sum(-1)
    ss = selfs.reshape(B, N, rep)[:, :, 0]
    scnt = jnp.sum((eidx == jnp.arange(N, dtype=jnp.int32).reshape(1, N, 1))
                   .astype(jnp.float32), axis=-1)
    npair = 10.0 * _KP * _LG
    return 0.5 * (ms + npair) - scnt * (0.5 * (ss + 10.0 * _LG))


# speedup vs baseline: 1.3696x; 1.3696x over previous
import functools
import numpy as np
import jax
import jax.numpy as jnp
from jax import lax
from jax.experimental import pallas as pl
from jax.experimental.pallas import tpu as pltpu
from jax.experimental.pallas import tpu_sc as plsc

_NUM_ATOMS = np.array([5, 11, 8, 8, 6, 9, 9, 4, 10, 8, 8, 9, 8, 11, 7, 6, 7, 14, 12, 7],
                      dtype=np.float32)
_SC_ELEMS = ["C", "CCCNCNN", "CCON", "CCOO", "CS", "CCCON", "CCCOO", "", "CCNCCN",
             "CCCC", "CCCC", "CCCCN", "CCSC", "CCCCCCC", "CCC", "CO", "COC",
             "CCCCNCCCCC", "CCCCCCCO", "CCC"]
_VDW = {"C": 1.7, "N": 1.55, "O": 1.52, "S": 1.8}


def _build_vdw_table():
    R = np.zeros((20, 14), dtype=np.float32)
    for i, sc in enumerate(_SC_ELEMS):
        for j, e in enumerate("NCCO" + sc):
            R[i, j] = _VDW[e]
    return R


_VDW_R = _build_vdw_table()

_LG = 16
_KP = 32
_RB = 256
_IW = 128
_CH_E = 128
_EPS4 = 0.001 / 4.0
_NEG = -30.0


def _sc_gather4(tables, gidx, n_edges):
    info = plsc.get_sparse_core_info()
    nc, ns = info.num_cores, info.num_subcores
    nw = nc * ns
    idx_rows = gidx.shape[1]
    e_per_w = idx_rows * _IW
    chunks = e_per_w // _CH_E
    q_per_ch = _CH_E // _IW
    mesh = plsc.VectorSubcoreMesh(core_axis_name="c", subcore_axis_name="s")
    out_t = jax.ShapeDtypeStruct((n_edges, _LG), jnp.float32)
    buf_t = pltpu.VMEM((_CH_E, _LG), jnp.float32)

    @functools.partial(
        pl.kernel,
        mesh=mesh,
        compiler_params=pltpu.CompilerParams(use_tc_tiling_on_sc=False),
        out_type=(out_t,) * 4,
        scratch_types=[
            pltpu.VMEM((idx_rows, _IW), jnp.int32),
            buf_t, buf_t, buf_t, buf_t,
            pltpu.SemaphoreType.DMA,
            pltpu.SemaphoreType.DMA,
            pltpu.SemaphoreType.DMA,
            pltpu.SemaphoreType.DMA,
        ],
    )
    def gather_kernel(t0, t1, t2, t3, gidx_hbm, o0, o1, o2, o3,
                      idx_v, r0, r1, r2, r3, s0, s1, s2, s3):
        wid = lax.axis_index("s") * nc + lax.axis_index("c")
        e_base = wid * e_per_w
        pltpu.sync_copy(gidx_hbm.at[wid], idx_v)
        tabs = ((t0, r0, s0), (t1, r1, s1), (t2, r2, s2), (t3, r3, s3))
        outs = (o0, o1, o2, o3)

        def body(c, carry):
            cps = []
            for (t, r, s) in tabs:
                for q in range(q_per_ch):
                    cps.append(pltpu.async_copy(
                        t.at[idx_v.at[c * q_per_ch + q]],
                        r.at[pl.ds(q * _IW, _IW)], s))
            for cp in cps:
                cp.wait()
            dst = pl.ds(e_base + c * _CH_E, _CH_E)
            for (t, r, s), o in zip(tabs, outs):
                pltpu.sync_copy(r, o.at[dst])
            return carry

        lax.fori_loop(0, chunks, body, 0)

    return gather_kernel(tables[0], tables[1], tables[2], tables[3], gidx)


def _tc_body(gx0_ref, gx1_ref, gx2_ref, gr_ref, ssq_ref, main_ref):
    g0, g1, g2, gr = gx0_ref[...], gx1_ref[...], gx2_ref[...], gr_ref[...]
    s = ssq_ref[...]

    acc = jnp.zeros_like(g0)
    for a in range(4, 14):
        qx0 = s[:, a:a + 1]
        qx1 = s[:, _LG + a:_LG + a + 1]
        qx2 = s[:, 2 * _LG + a:2 * _LG + a + 1]
        qr = s[:, 3 * _LG + a:3 * _LG + a + 1]
        d2 = (qx0 - g0) ** 2 + ((qx1 - g1) ** 2 + ((qx2 - g2) ** 2 + _EPS4))
        acc = acc + jnp.tanh((qr + gr) - jnp.sqrt(d2))

    main_ref[...] = jnp.sum(acc, axis=1, keepdims=True)


def _tc_compute(gx, ssq, n_rows, interpret=False):
    rows_b = _RB * (_KP * _LG // 128)
    grid = (n_rows // rows_b,)
    gspec = pl.BlockSpec((rows_b, 128), lambda i: (i, 0))
    sspec = pl.BlockSpec((rows_b, 4 * _LG), lambda i: (i, 0))
    ospec = pl.BlockSpec((rows_b, 1), lambda i: (i, 0))
    o_t = jax.ShapeDtypeStruct((n_rows, 1), jnp.float32)
    return pl.pallas_call(
        _tc_body,
        grid=grid,
        in_specs=[gspec] * 4 + [sspec],
        out_specs=ospec,
        out_shape=o_t,
        interpret=interpret,
    )(*gx, ssq)


def _build_tables(X, C, S):
    B, N, A, _ = X.shape
    onehot = (S[:, :, None] == jnp.arange(20, dtype=S.dtype)).astype(jnp.float32)
    rmat = jnp.asarray(_VDW_R * 0.25 + 0.0875)
    rrow = jnp.dot(onehot, rmat, precision=jax.lax.Precision.HIGHEST)
    apr = (C > 0).astype(jnp.float32) * jnp.dot(onehot, jnp.asarray(_NUM_ATOMS),
                                                precision=jax.lax.Precision.HIGHEST)
    mask = jnp.arange(A, dtype=jnp.float32).reshape(1, 1, A) < apr[:, :, None]
    r4 = jnp.where(mask, rrow, _NEG)
    pad0 = jnp.zeros((B, N, _LG - A), jnp.float32)
    padn = jnp.full((B, N, _LG - A), _NEG, jnp.float32)
    xh = X * 0.5
    zrow = jnp.zeros((1, _LG), jnp.float32)
    nrow = jnp.full((1, _LG), _NEG, jnp.float32)
    tx0 = jnp.concatenate(
        [jnp.concatenate([xh[:, :, :, 0], pad0], -1).reshape(B * N, _LG), zrow], 0)
    tx1 = jnp.concatenate(
        [jnp.concatenate([xh[:, :, :, 1], pad0], -1).reshape(B * N, _LG), zrow], 0)
    tx2 = jnp.concatenate(
        [jnp.concatenate([xh[:, :, :, 2], pad0], -1).reshape(B * N, _LG), zrow], 0)
    tr = jnp.concatenate(
        [jnp.concatenate([r4, padn], -1).reshape(B * N, _LG), nrow], 0)
    return (tx0, tx1, tx2, tr)


def kernel(X, C, S, edge_idx):
    B, N, A, _ = X.shape
    Kn = edge_idx.shape[2]
    n_edges = B * N * _KP
    rep = _KP * _LG // 128

    tables = _build_tables(X, C, S)

    eidx = edge_idx.astype(jnp.int32)
    gidx = eidx + (jnp.arange(B, dtype=jnp.int32) * N)[:, None, None]
    gidx = jnp.concatenate(
        [gidx, jnp.full((B, N, _KP - Kn), B * N, jnp.int32)], axis=-1).reshape(-1)
    nw = 32
    gidx3 = gidx.reshape(nw, n_edges // (nw * _IW), _IW)

    g = _sc_gather4(tables, gidx3, n_edges)
    gx = tuple(t.reshape(n_edges * _LG // 128, 128) for t in g)
    ssq = jnp.repeat(jnp.concatenate([t[:B * N] for t in tables], axis=-1),
                     rep, axis=0)

    main = _tc_compute(gx, ssq, B * N * rep)
    ms = main.reshape(B, N, rep).sum(-1)

    sx = tuple(t[:B * N].reshape(B, N, _LG) for t in tables)
    ds2 = sum((s[:, :, 4:A, None] - s[:, :, None, :]) ** 2 for s in sx[:3]) + _EPS4
    args = (sx[3][:, :, 4:A, None] + sx[3][:, :, None, :]) - jnp.sqrt(ds2)
    ss = jnp.sum(jnp.tanh(args), axis=(-1, -2))

    scnt = jnp.sum((eidx == jnp.arange(N, dtype=jnp.int32).reshape(1, N, 1))
                   .astype(jnp.float32), axis=-1)
    npair = 10.0 * _KP * _LG
    return 0.5 * (ms + npair) - scnt * (0.5 * (ss + 10.0 * _LG))


# speedup vs baseline: 1.4649x; 1.0696x over previous
import functools
import numpy as np
import jax
import jax.numpy as jnp
from jax import lax
from jax.experimental import pallas as pl
from jax.experimental.pallas import tpu as pltpu
from jax.experimental.pallas import tpu_sc as plsc

_NUM_ATOMS = np.array([5, 11, 8, 8, 6, 9, 9, 4, 10, 8, 8, 9, 8, 11, 7, 6, 7, 14, 12, 7],
                      dtype=np.float32)
_SC_ELEMS = ["C", "CCCNCNN", "CCON", "CCOO", "CS", "CCCON", "CCCOO", "", "CCNCCN",
             "CCCC", "CCCC", "CCCCN", "CCSC", "CCCCCCC", "CCC", "CO", "COC",
             "CCCCNCCCCC", "CCCCCCCO", "CCC"]
_VDW = {"C": 1.7, "N": 1.55, "O": 1.52, "S": 1.8}


def _build_vdw_table():
    R = np.zeros((20, 14), dtype=np.float32)
    for i, sc in enumerate(_SC_ELEMS):
        for j, e in enumerate("NCCO" + sc):
            R[i, j] = _VDW[e]
    return R


_VDW_R = _build_vdw_table()

_LG = 16
_KP = 32
_RB = 256
_IW = 128
_CH_E = 128
_EPS4 = 0.001 / 4.0
_NEG = -30.0
_NPOIS = 64


def _sc_gather4(tables, gidx, n_edges):
    info = plsc.get_sparse_core_info()
    nc, ns = info.num_cores, info.num_subcores
    nw = nc * ns
    idx_rows = gidx.shape[1]
    e_per_w = idx_rows * _IW
    chunks = e_per_w // _CH_E
    q_per_ch = _CH_E // _IW
    mesh = plsc.VectorSubcoreMesh(core_axis_name="c", subcore_axis_name="s")
    out_t = jax.ShapeDtypeStruct((n_edges, _LG), jnp.float32)
    buf_t = pltpu.VMEM((_CH_E, _LG), jnp.float32)

    @functools.partial(
        pl.kernel,
        mesh=mesh,
        compiler_params=pltpu.CompilerParams(use_tc_tiling_on_sc=False),
        out_type=(out_t,) * 4,
        scratch_types=[
            pltpu.VMEM((idx_rows, _IW), jnp.int32),
            buf_t, buf_t, buf_t, buf_t,
            pltpu.SemaphoreType.DMA,
            pltpu.SemaphoreType.DMA,
            pltpu.SemaphoreType.DMA,
            pltpu.SemaphoreType.DMA,
        ],
    )
    def gather_kernel(t0, t1, t2, t3, gidx_hbm, o0, o1, o2, o3,
                      idx_v, r0, r1, r2, r3, s0, s1, s2, s3):
        wid = lax.axis_index("s") * nc + lax.axis_index("c")
        e_base = wid * e_per_w
        pltpu.sync_copy(gidx_hbm.at[wid], idx_v)
        tabs = ((t0, r0, s0), (t1, r1, s1), (t2, r2, s2), (t3, r3, s3))
        outs = (o0, o1, o2, o3)

        def body(c, carry):
            cps = []
            for (t, r, s) in tabs:
                for q in range(q_per_ch):
                    cps.append(pltpu.async_copy(
                        t.at[idx_v.at[c * q_per_ch + q]],
                        r.at[pl.ds(q * _IW, _IW)], s))
            for cp in cps:
                cp.wait()
            dst = pl.ds(e_base + c * _CH_E, _CH_E)
            for (t, r, s), o in zip(tabs, outs):
                pltpu.sync_copy(r, o.at[dst])
            return carry

        lax.fori_loop(0, chunks, body, 0)

    return gather_kernel(tables[0], tables[1], tables[2], tables[3], gidx)


def _tc_body(gx0_ref, gx1_ref, gx2_ref, gr_ref, ssq_ref, main_ref):
    g0, g1, g2, gr = gx0_ref[...], gx1_ref[...], gx2_ref[...], gr_ref[...]
    s = ssq_ref[...]

    acc = jnp.zeros_like(g0)
    for a in range(4, 14):
        qx0 = s[:, a:a + 1]
        qx1 = s[:, _LG + a:_LG + a + 1]
        qx2 = s[:, 2 * _LG + a:2 * _LG + a + 1]
        qr = s[:, 3 * _LG + a:3 * _LG + a + 1]
        d2 = (qx0 - g0) ** 2 + ((qx1 - g1) ** 2 + ((qx2 - g2) ** 2 + _EPS4))
        acc = acc + jnp.tanh((qr + gr) - jnp.sqrt(d2))

    main_ref[...] = jnp.sum(acc, axis=1, keepdims=True)


def _tc_compute(gx, ssq, n_rows, interpret=False):
    rows_b = _RB * (_KP * _LG // 128)
    grid = (n_rows // rows_b,)
    gspec = pl.BlockSpec((rows_b, 128), lambda i: (i, 0))
    sspec = pl.BlockSpec((rows_b, 4 * _LG), lambda i: (i, 0))
    ospec = pl.BlockSpec((rows_b, 1), lambda i: (i, 0))
    o_t = jax.ShapeDtypeStruct((n_rows, 1), jnp.float32)
    return pl.pallas_call(
        _tc_body,
        grid=grid,
        in_specs=[gspec] * 4 + [sspec],
        out_specs=ospec,
        out_shape=o_t,
        interpret=interpret,
    )(*gx, ssq)


def _build_tables(X, C, S):
    B, N, A, _ = X.shape
    onehot = (S[:, :, None] == jnp.arange(20, dtype=S.dtype)).astype(jnp.float32)
    rmat = jnp.asarray(_VDW_R * 0.25 + 0.0875)
    rrow = jnp.dot(onehot, rmat, precision=jax.lax.Precision.HIGHEST)
    apr = (C > 0).astype(jnp.float32) * jnp.dot(onehot, jnp.asarray(_NUM_ATOMS),
                                                precision=jax.lax.Precision.HIGHEST)
    mask = jnp.arange(A, dtype=jnp.float32).reshape(1, 1, A) < apr[:, :, None]
    r4 = jnp.where(mask, rrow, _NEG)
    pad0 = jnp.zeros((B, N, _LG - A), jnp.float32)
    padn = jnp.full((B, N, _LG - A), _NEG, jnp.float32)
    xh = X * 0.5
    zrow = jnp.zeros((_NPOIS, _LG), jnp.float32)
    nrow = jnp.full((_NPOIS, _LG), _NEG, jnp.float32)
    tx0 = jnp.concatenate(
        [jnp.concatenate([xh[:, :, :, 0], pad0], -1).reshape(B * N, _LG), zrow], 0)
    tx1 = jnp.concatenate(
        [jnp.concatenate([xh[:, :, :, 1], pad0], -1).reshape(B * N, _LG), zrow], 0)
    tx2 = jnp.concatenate(
        [jnp.concatenate([xh[:, :, :, 2], pad0], -1).reshape(B * N, _LG), zrow], 0)
    tr = jnp.concatenate(
        [jnp.concatenate([r4, padn], -1).reshape(B * N, _LG), nrow], 0)
    return (tx0, tx1, tx2, tr)


def kernel(X, C, S, edge_idx):
    B, N, A, _ = X.shape
    Kn = edge_idx.shape[2]
    n_edges = B * N * _KP
    rep = _KP * _LG // 128

    tables = _build_tables(X, C, S)

    eidx = edge_idx.astype(jnp.int32)
    gidx = eidx + (jnp.arange(B, dtype=jnp.int32) * N)[:, None, None]
    pvals = (B * N + (jnp.arange(B * N * (_KP - Kn), dtype=jnp.int32) % _NPOIS)
             ).reshape(B, N, _KP - Kn)
    gidx = jnp.concatenate([gidx, pvals], axis=-1).reshape(-1)
    nw = 32
    gidx3 = gidx.reshape(nw, n_edges // (nw * _IW), _IW)

    g = _sc_gather4(tables, gidx3, n_edges)
    gx = tuple(t.reshape(n_edges * _LG // 128, 128) for t in g)
    ssq = jnp.repeat(jnp.concatenate([t[:B * N] for t in tables], axis=-1),
                     rep, axis=0)

    main = _tc_compute(gx, ssq, B * N * rep)
    ms = main.reshape(B, N, rep).sum(-1)

    sx = tuple(t[:B * N].reshape(B, N, _LG) for t in tables)
    ds2 = sum((s[:, :, 4:A, None] - s[:, :, None, :]) ** 2 for s in sx[:3]) + _EPS4
    args = (sx[3][:, :, 4:A, None] + sx[3][:, :, None, :]) - jnp.sqrt(ds2)
    ss = jnp.sum(jnp.tanh(args), axis=(-1, -2))

    scnt = jnp.sum((eidx == jnp.arange(N, dtype=jnp.int32).reshape(1, N, 1))
                   .astype(jnp.float32), axis=-1)
    npair = 10.0 * _KP * _LG
    return 0.5 * (ms + npair) - scnt * (0.5 * (ss + 10.0 * _LG))


# speedup vs baseline: 1.4703x; 1.0037x over previous
import functools
import numpy as np
import jax
import jax.numpy as jnp
from jax import lax
from jax.experimental import pallas as pl
from jax.experimental.pallas import tpu as pltpu
from jax.experimental.pallas import tpu_sc as plsc

_NUM_ATOMS = np.array([5, 11, 8, 8, 6, 9, 9, 4, 10, 8, 8, 9, 8, 11, 7, 6, 7, 14, 12, 7],
                      dtype=np.float32)
_SC_ELEMS = ["C", "CCCNCNN", "CCON", "CCOO", "CS", "CCCON", "CCCOO", "", "CCNCCN",
             "CCCC", "CCCC", "CCCCN", "CCSC", "CCCCCCC", "CCC", "CO", "COC",
             "CCCCNCCCCC", "CCCCCCCO", "CCC"]
_VDW = {"C": 1.7, "N": 1.55, "O": 1.52, "S": 1.8}


def _build_vdw_table():
    R = np.zeros((20, 14), dtype=np.float32)
    for i, sc in enumerate(_SC_ELEMS):
        for j, e in enumerate("NCCO" + sc):
            R[i, j] = _VDW[e]
    return R


_VDW_R = _build_vdw_table()

_LG = 16
_KP = 32
_RB = 128
_IW = 128
_CH_E = 128
_EPS4 = 0.001 / 4.0
_NEG = -30.0
_NPOIS = 64


def _sc_gather4(tables, gidx, n_edges):
    info = plsc.get_sparse_core_info()
    nc, ns = info.num_cores, info.num_subcores
    nw = nc * ns
    idx_rows = gidx.shape[1]
    e_per_w = idx_rows * _IW
    chunks = e_per_w // _CH_E
    q_per_ch = _CH_E // _IW
    mesh = plsc.VectorSubcoreMesh(core_axis_name="c", subcore_axis_name="s")
    out_t = jax.ShapeDtypeStruct((n_edges, _LG), jnp.float32)
    buf_t = pltpu.VMEM((_CH_E, _LG), jnp.float32)

    @functools.partial(
        pl.kernel,
        mesh=mesh,
        compiler_params=pltpu.CompilerParams(use_tc_tiling_on_sc=False),
        out_type=(out_t,) * 4,
        scratch_types=[
            pltpu.VMEM((idx_rows, _IW), jnp.int32),
            buf_t, buf_t, buf_t, buf_t,
            pltpu.SemaphoreType.DMA,
            pltpu.SemaphoreType.DMA,
            pltpu.SemaphoreType.DMA,
            pltpu.SemaphoreType.DMA,
        ],
    )
    def gather_kernel(t0, t1, t2, t3, gidx_hbm, o0, o1, o2, o3,
                      idx_v, r0, r1, r2, r3, s0, s1, s2, s3):
        wid = lax.axis_index("s") * nc + lax.axis_index("c")
        e_base = wid * e_per_w
        pltpu.sync_copy(gidx_hbm.at[wid], idx_v)
        tabs = ((t0, r0, s0), (t1, r1, s1), (t2, r2, s2), (t3, r3, s3))
        outs = (o0, o1, o2, o3)

        def body(c, carry):
            cps = []
            for (t, r, s) in tabs:
                for q in range(q_per_ch):
                    cps.append(pltpu.async_copy(
                        t.at[idx_v.at[c * q_per_ch + q]],
                        r.at[pl.ds(q * _IW, _IW)], s))
            for cp in cps:
                cp.wait()
            dst = pl.ds(e_base + c * _CH_E, _CH_E)
            for (t, r, s), o in zip(tabs, outs):
                pltpu.sync_copy(r, o.at[dst])
            return carry

        lax.fori_loop(0, chunks, body, 0)

    return gather_kernel(tables[0], tables[1], tables[2], tables[3], gidx)


def _tc_body(gx0_ref, gx1_ref, gx2_ref, gr_ref, ssq_ref, main_ref):
    g0, g1, g2, gr = gx0_ref[...], gx1_ref[...], gx2_ref[...], gr_ref[...]
    s = ssq_ref[...]

    acc = jnp.zeros_like(g0)
    for a in range(4, 14):
        qx0 = s[:, a:a + 1]
        qx1 = s[:, _LG + a:_LG + a + 1]
        qx2 = s[:, 2 * _LG + a:2 * _LG + a + 1]
        qr = s[:, 3 * _LG + a:3 * _LG + a + 1]
        d2 = (qx0 - g0) ** 2 + ((qx1 - g1) ** 2 + ((qx2 - g2) ** 2 + _EPS4))
        acc = acc + jnp.tanh((qr + gr) - jnp.sqrt(d2))

    main_ref[...] = jnp.sum(acc, axis=1, keepdims=True)


def _tc_compute(gx, ssq, n_rows, interpret=False):
    rows_b = _RB * (_KP * _LG // 128)
    grid = (n_rows // rows_b,)
    gspec = pl.BlockSpec((rows_b, 128), lambda i: (i, 0))
    sspec = pl.BlockSpec((rows_b, 4 * _LG), lambda i: (i, 0))
    ospec = pl.BlockSpec((rows_b, 1), lambda i: (i, 0))
    o_t = jax.ShapeDtypeStruct((n_rows, 1), jnp.float32)
    return pl.pallas_call(
        _tc_body,
        grid=grid,
        in_specs=[gspec] * 4 + [sspec],
        out_specs=ospec,
        out_shape=o_t,
        interpret=interpret,
    )(*gx, ssq)


def _build_tables(X, C, S):
    B, N, A, _ = X.shape
    onehot = (S[:, :, None] == jnp.arange(20, dtype=S.dtype)).astype(jnp.float32)
    rmat = jnp.asarray(_VDW_R * 0.25 + 0.0875)
    rrow = jnp.dot(onehot, rmat, precision=jax.lax.Precision.HIGHEST)
    apr = (C > 0).astype(jnp.float32) * jnp.dot(onehot, jnp.asarray(_NUM_ATOMS),
                                                precision=jax.lax.Precision.HIGHEST)
    mask = jnp.arange(A, dtype=jnp.float32).reshape(1, 1, A) < apr[:, :, None]
    r4 = jnp.where(mask, rrow, _NEG)
    pad0 = jnp.zeros((B, N, _LG - A), jnp.float32)
    padn = jnp.full((B, N, _LG - A), _NEG, jnp.float32)
    xh = X * 0.5
    zrow = jnp.zeros((_NPOIS, _LG), jnp.float32)
    nrow = jnp.full((_NPOIS, _LG), _NEG, jnp.float32)
    tx0 = jnp.concatenate(
        [jnp.concatenate([xh[:, :, :, 0], pad0], -1).reshape(B * N, _LG), zrow], 0)
    tx1 = jnp.concatenate(
        [jnp.concatenate([xh[:, :, :, 1], pad0], -1).reshape(B * N, _LG), zrow], 0)
    tx2 = jnp.concatenate(
        [jnp.concatenate([xh[:, :, :, 2], pad0], -1).reshape(B * N, _LG), zrow], 0)
    tr = jnp.concatenate(
        [jnp.concatenate([r4, padn], -1).reshape(B * N, _LG), nrow], 0)
    return (tx0, tx1, tx2, tr)


def kernel(X, C, S, edge_idx):
    B, N, A, _ = X.shape
    Kn = edge_idx.shape[2]
    n_edges = B * N * _KP
    rep = _KP * _LG // 128

    tables = _build_tables(X, C, S)

    eidx = edge_idx.astype(jnp.int32)
    gidx = eidx + (jnp.arange(B, dtype=jnp.int32) * N)[:, None, None]
    pvals = (B * N + (jnp.arange(B * N * (_KP - Kn), dtype=jnp.int32) % _NPOIS)
             ).reshape(B, N, _KP - Kn)
    gidx = jnp.concatenate([gidx, pvals], axis=-1).reshape(-1)
    nw = 32
    gidx3 = gidx.reshape(nw, n_edges // (nw * _IW), _IW)

    g = _sc_gather4(tables, gidx3, n_edges)
    gx = tuple(t.reshape(n_edges * _LG // 128, 128) for t in g)
    ssq = jnp.repeat(jnp.concatenate([t[:B * N] for t in tables], axis=-1),
                     rep, axis=0)

    main = _tc_compute(gx, ssq, B * N * rep)
    ms = main.reshape(B, N, rep).sum(-1)

    sx = tuple(t[:B * N].reshape(B, N, _LG) for t in tables)
    ds2 = sum((s[:, :, 4:A, None] - s[:, :, None, :]) ** 2 for s in sx[:3]) + _EPS4
    args = (sx[3][:, :, 4:A, None] + sx[3][:, :, None, :]) - jnp.sqrt(ds2)
    ss = jnp.sum(jnp.tanh(args), axis=(-1, -2))

    scnt = jnp.sum((eidx == jnp.arange(N, dtype=jnp.int32).reshape(1, N, 1))
                   .astype(jnp.float32), axis=-1)
    npair = 10.0 * _KP * _LG
    return 0.5 * (ms + npair) - scnt * (0.5 * (ss + 10.0 * _LG))


# speedup vs baseline: 1.6392x; 1.1149x over previous
import functools
import numpy as np
import jax
import jax.numpy as jnp
from jax import lax
from jax.experimental import pallas as pl
from jax.experimental.pallas import tpu as pltpu
from jax.experimental.pallas import tpu_sc as plsc

_NUM_ATOMS = np.array([5, 11, 8, 8, 6, 9, 9, 4, 10, 8, 8, 9, 8, 11, 7, 6, 7, 14, 12, 7],
                      dtype=np.float32)
_SC_ELEMS = ["C", "CCCNCNN", "CCON", "CCOO", "CS", "CCCON", "CCCOO", "", "CCNCCN",
             "CCCC", "CCCC", "CCCCN", "CCSC", "CCCCCCC", "CCC", "CO", "COC",
             "CCCCNCCCCC", "CCCCCCCO", "CCC"]
_VDW = {"C": 1.7, "N": 1.55, "O": 1.52, "S": 1.8}


def _build_vdw_table():
    R = np.zeros((20, 14), dtype=np.float32)
    for i, sc in enumerate(_SC_ELEMS):
        for j, e in enumerate("NCCO" + sc):
            R[i, j] = _VDW[e]
    return R


_VDW_R = _build_vdw_table()

_LG = 16
_KP = 32
_RB = 128
_IW = 128
_CH_E = 128
_EPS4 = 0.001 / 4.0
_NEG = -30.0
_NPOIS = 64


def _sc_gather4(tables, gidx, n_edges):
    info = plsc.get_sparse_core_info()
    nc, ns = info.num_cores, info.num_subcores
    nw = nc * ns
    idx_rows = gidx.shape[1]
    e_per_w = idx_rows * _IW
    chunks = e_per_w // _CH_E
    q_per_ch = _CH_E // _IW
    mesh = plsc.VectorSubcoreMesh(core_axis_name="c", subcore_axis_name="s")
    out_t = jax.ShapeDtypeStruct((n_edges, _LG), jnp.float32)
    buf_t = pltpu.VMEM((_CH_E, _LG), jnp.float32)

    @functools.partial(
        pl.kernel,
        mesh=mesh,
        compiler_params=pltpu.CompilerParams(use_tc_tiling_on_sc=False),
        out_type=(out_t,) * 4,
        scratch_types=[
            pltpu.VMEM((idx_rows, _IW), jnp.int32),
            buf_t, buf_t, buf_t, buf_t,
            pltpu.SemaphoreType.DMA,
            pltpu.SemaphoreType.DMA,
            pltpu.SemaphoreType.DMA,
            pltpu.SemaphoreType.DMA,
        ],
    )
    def gather_kernel(t0, t1, t2, t3, gidx_hbm, o0, o1, o2, o3,
                      idx_v, r0, r1, r2, r3, s0, s1, s2, s3):
        wid = lax.axis_index("s") * nc + lax.axis_index("c")
        e_base = wid * e_per_w
        pltpu.sync_copy(gidx_hbm.at[wid], idx_v)
        tabs = ((t0, r0, s0), (t1, r1, s1), (t2, r2, s2), (t3, r3, s3))
        outs = (o0, o1, o2, o3)

        def body(c, carry):
            cps = []
            for (t, r, s) in tabs:
                for q in range(q_per_ch):
                    cps.append(pltpu.async_copy(
                        t.at[idx_v.at[c * q_per_ch + q]],
                        r.at[pl.ds(q * _IW, _IW)], s))
            for cp in cps:
                cp.wait()
            dst = pl.ds(e_base + c * _CH_E, _CH_E)
            for (t, r, s), o in zip(tabs, outs):
                pltpu.sync_copy(r, o.at[dst])
            return carry

        lax.fori_loop(0, chunks, body, 0)

    return gather_kernel(tables[0], tables[1], tables[2], tables[3], gidx)


def _tc_body(gx0_ref, gx1_ref, gx2_ref, gr_ref, ssq_ref, main_ref):
    g0, g1, g2, gr = gx0_ref[...], gx1_ref[...], gx2_ref[...], gr_ref[...]
    s = ssq_ref[...]

    acc = jnp.zeros_like(g0)
    for a in range(4, 14):
        qx0 = s[:, a:a + 1]
        qx1 = s[:, _LG + a:_LG + a + 1]
        qx2 = s[:, 2 * _LG + a:2 * _LG + a + 1]
        qr = s[:, 3 * _LG + a:3 * _LG + a + 1]
        d2 = (qx0 - g0) ** 2 + ((qx1 - g1) ** 2 + ((qx2 - g2) ** 2 + _EPS4))
        acc = acc + jnp.tanh((qr + gr) - jnp.sqrt(d2))

    main_ref[...] = jnp.sum(acc, axis=1, keepdims=True)


def _tc_compute(gx, ssq, n_rows, interpret=False):
    rows_b = _RB * (_KP * _LG // 128)
    grid = (n_rows // rows_b,)
    gspec = pl.BlockSpec((rows_b, 128), lambda i: (i, 0))
    sspec = pl.BlockSpec((rows_b, 4 * _LG), lambda i: (i, 0))
    ospec = pl.BlockSpec((rows_b, 1), lambda i: (i, 0))
    o_t = jax.ShapeDtypeStruct((n_rows, 1), jnp.float32)
    return pl.pallas_call(
        _tc_body,
        grid=grid,
        in_specs=[gspec] * 4 + [sspec],
        out_specs=ospec,
        out_shape=o_t,
        interpret=interpret,
    )(*gx, ssq)


def _build_tables(X, C, S):
    B, N, A, _ = X.shape
    onehot = (S[:, :, None] == jnp.arange(20, dtype=S.dtype)).astype(jnp.float32)
    rmat = jnp.asarray(_VDW_R * 0.25 + 0.0875)
    rrow = jnp.dot(onehot, rmat, precision=jax.lax.Precision.HIGHEST)
    apr = (C > 0).astype(jnp.float32) * jnp.dot(onehot, jnp.asarray(_NUM_ATOMS),
                                                precision=jax.lax.Precision.HIGHEST)
    mask = jnp.arange(A, dtype=jnp.float32).reshape(1, 1, A) < apr[:, :, None]
    r4 = jnp.where(mask, rrow, _NEG)
    pad0 = jnp.zeros((B, N, _LG - A), jnp.float32)
    padn = jnp.full((B, N, _LG - A), _NEG, jnp.float32)
    xh = X * 0.5
    zrow = jnp.zeros((_NPOIS, _LG), jnp.float32)
    nrow = jnp.full((_NPOIS, _LG), _NEG, jnp.float32)
    tx0 = jnp.concatenate(
        [jnp.concatenate([xh[:, :, :, 0], pad0], -1).reshape(B * N, _LG), zrow], 0)
    tx1 = jnp.concatenate(
        [jnp.concatenate([xh[:, :, :, 1], pad0], -1).reshape(B * N, _LG), zrow], 0)
    tx2 = jnp.concatenate(
        [jnp.concatenate([xh[:, :, :, 2], pad0], -1).reshape(B * N, _LG), zrow], 0)
    tr = jnp.concatenate(
        [jnp.concatenate([r4, padn], -1).reshape(B * N, _LG), nrow], 0)
    return (tx0, tx1, tx2, tr)


def kernel(X, C, S, edge_idx):
    B, N, A, _ = X.shape
    Kn = edge_idx.shape[2]
    n_edges = B * N * _KP
    rep = _KP * _LG // 128

    tables = _build_tables(X, C, S)

    eidx = edge_idx.astype(jnp.int32)
    gidx = eidx + (jnp.arange(B, dtype=jnp.int32) * N)[:, None, None]
    pvals = (B * N + (jnp.arange(B * N * (_KP - Kn), dtype=jnp.int32) % _NPOIS)
             ).reshape(B, N, _KP - Kn)
    gidx = jnp.concatenate([gidx, pvals], axis=-1)
    nw = 32
    ne_b = N * _KP

    ms_parts = []
    for b in range(B):
        gidx3 = gidx[b].reshape(nw, ne_b // (nw * _IW), _IW)
        g = _sc_gather4(tables, gidx3, ne_b)
        gxb = tuple(t.reshape(ne_b * _LG // 128, 128) for t in g)
        ssqb = jnp.repeat(
            jnp.concatenate([t[b * N:(b + 1) * N] for t in tables], axis=-1),
            rep, axis=0)
        main = _tc_compute(gxb, ssqb, N * rep)
        ms_parts.append(main.reshape(N, rep).sum(-1))
    ms = jnp.stack(ms_parts)

    sx = tuple(t[:B * N].reshape(B, N, _LG) for t in tables)
    ds2 = sum((s[:, :, 4:A, None] - s[:, :, None, :]) ** 2 for s in sx[:3]) + _EPS4
    args = (sx[3][:, :, 4:A, None] + sx[3][:, :, None, :]) - jnp.sqrt(ds2)
    ss = jnp.sum(jnp.tanh(args), axis=(-1, -2))

    scnt = jnp.sum((eidx == jnp.arange(N, dtype=jnp.int32).reshape(1, N, 1))
                   .astype(jnp.float32), axis=-1)
    npair = 10.0 * _KP * _LG
    return 0.5 * (ms + npair) - scnt * (0.5 * (ss + 10.0 * _LG))


# speedup vs baseline: 1.6524x; 1.0081x over previous
import functools
import numpy as np
import jax
import jax.numpy as jnp
from jax import lax
from jax.experimental import pallas as pl
from jax.experimental.pallas import tpu as pltpu
from jax.experimental.pallas import tpu_sc as plsc

_NUM_ATOMS = np.array([5, 11, 8, 8, 6, 9, 9, 4, 10, 8, 8, 9, 8, 11, 7, 6, 7, 14, 12, 7],
                      dtype=np.float32)
_SC_ELEMS = ["C", "CCCNCNN", "CCON", "CCOO", "CS", "CCCON", "CCCOO", "", "CCNCCN",
             "CCCC", "CCCC", "CCCCN", "CCSC", "CCCCCCC", "CCC", "CO", "COC",
             "CCCCNCCCCC", "CCCCCCCO", "CCC"]
_VDW = {"C": 1.7, "N": 1.55, "O": 1.52, "S": 1.8}


def _build_vdw_table():
    R = np.zeros((20, 14), dtype=np.float32)
    for i, sc in enumerate(_SC_ELEMS):
        for j, e in enumerate("NCCO" + sc):
            R[i, j] = _VDW[e]
    return R


_VDW_R = _build_vdw_table()

_LG = 16
_KP = 32
_RB = 128
_IW = 128
_CH_E = 128
_EPS4 = 0.001 / 4.0
_NEG = -30.0
_NPOIS = 64


def _sc_gather4(tables, gidx, n_edges):
    info = plsc.get_sparse_core_info()
    nc, ns = info.num_cores, info.num_subcores
    nw = nc * ns
    idx_rows = gidx.shape[1]
    e_per_w = idx_rows * _IW
    chunks = e_per_w // _CH_E
    q_per_ch = _CH_E // _IW
    mesh = plsc.VectorSubcoreMesh(core_axis_name="c", subcore_axis_name="s")
    out_t = jax.ShapeDtypeStruct((n_edges, _LG), jnp.float32)
    buf_t = pltpu.VMEM((_CH_E, _LG), jnp.float32)

    @functools.partial(
        pl.kernel,
        mesh=mesh,
        compiler_params=pltpu.CompilerParams(use_tc_tiling_on_sc=False),
        out_type=(out_t,) * 4,
        scratch_types=[
            pltpu.VMEM((idx_rows, _IW), jnp.int32),
            buf_t, buf_t, buf_t, buf_t,
            pltpu.SemaphoreType.DMA,
            pltpu.SemaphoreType.DMA,
            pltpu.SemaphoreType.DMA,
            pltpu.SemaphoreType.DMA,
        ],
    )
    def gather_kernel(t0, t1, t2, t3, gidx_hbm, o0, o1, o2, o3,
                      idx_v, r0, r1, r2, r3, s0, s1, s2, s3):
        wid = lax.axis_index("s") * nc + lax.axis_index("c")
        e_base = wid * e_per_w
        pltpu.sync_copy(gidx_hbm.at[wid], idx_v)
        tabs = ((t0, r0, s0), (t1, r1, s1), (t2, r2, s2), (t3, r3, s3))
        outs = (o0, o1, o2, o3)

        def body(c, carry):
            cps = []
            for (t, r, s) in tabs:
                for q in range(q_per_ch):
                    cps.append(pltpu.async_copy(
                        t.at[idx_v.at[c * q_per_ch + q]],
                        r.at[pl.ds(q * _IW, _IW)], s))
            for cp in cps:
                cp.wait()
            dst = pl.ds(e_base + c * _CH_E, _CH_E)
            for (t, r, s), o in zip(tabs, outs):
                pltpu.sync_copy(r, o.at[dst])
            return carry

        lax.fori_loop(0, chunks, body, 0)

    return gather_kernel(tables[0], tables[1], tables[2], tables[3], gidx)


def _tc_body(gx0_ref, gx1_ref, gx2_ref, gr_ref, ssq_ref, main_ref):
    g0, g1, g2, gr = gx0_ref[...], gx1_ref[...], gx2_ref[...], gr_ref[...]
    s = ssq_ref[...]

    acc = jnp.zeros_like(g0)
    for a in range(4, 14):
        qx0 = s[:, a:a + 1]
        qx1 = s[:, _LG + a:_LG + a + 1]
        qx2 = s[:, 2 * _LG + a:2 * _LG + a + 1]
        qr = s[:, 3 * _LG + a:3 * _LG + a + 1]
        d2 = (qx0 - g0) ** 2 + ((qx1 - g1) ** 2 + ((qx2 - g2) ** 2 + _EPS4))
        acc = acc + jnp.tanh((qr + gr) - jnp.sqrt(d2))

    main_ref[...] = jnp.sum(acc, axis=1, keepdims=True)


def _tc_compute(gx, ssq, n_rows, interpret=False):
    rows_b = _RB * (_KP * _LG // 128)
    grid = (n_rows // rows_b,)
    gspec = pl.BlockSpec((rows_b, 128), lambda i: (i, 0))
    sspec = pl.BlockSpec((rows_b, 4 * _LG), lambda i: (i, 0))
    ospec = pl.BlockSpec((rows_b, 1), lambda i: (i, 0))
    o_t = jax.ShapeDtypeStruct((n_rows, 1), jnp.float32)
    return pl.pallas_call(
        _tc_body,
        grid=grid,
        in_specs=[gspec] * 4 + [sspec],
        out_specs=ospec,
        out_shape=o_t,
        interpret=interpret,
    )(*gx, ssq)


def _build_tables(X, C, S):
    B, N, A, _ = X.shape
    onehot = (S[:, :, None] == jnp.arange(20, dtype=S.dtype)).astype(jnp.float32)
    rmat = jnp.asarray(_VDW_R * 0.25 + 0.0875)
    rrow = jnp.dot(onehot, rmat, precision=jax.lax.Precision.HIGHEST)
    apr = (C > 0).astype(jnp.float32) * jnp.dot(onehot, jnp.asarray(_NUM_ATOMS),
                                                precision=jax.lax.Precision.HIGHEST)
    mask = jnp.arange(A, dtype=jnp.float32).reshape(1, 1, A) < apr[:, :, None]
    r4 = jnp.where(mask, rrow, _NEG)
    pad0 = jnp.zeros((B, N, _LG - A), jnp.float32)
    padn = jnp.full((B, N, _LG - A), _NEG, jnp.float32)
    xh = X * 0.5
    zrow = jnp.zeros((_NPOIS, _LG), jnp.float32)
    nrow = jnp.full((_NPOIS, _LG), _NEG, jnp.float32)
    tx0 = jnp.concatenate(
        [jnp.concatenate([xh[:, :, :, 0], pad0], -1).reshape(B * N, _LG), zrow], 0)
    tx1 = jnp.concatenate(
        [jnp.concatenate([xh[:, :, :, 1], pad0], -1).reshape(B * N, _LG), zrow], 0)
    tx2 = jnp.concatenate(
        [jnp.concatenate([xh[:, :, :, 2], pad0], -1).reshape(B * N, _LG), zrow], 0)
    tr = jnp.concatenate(
        [jnp.concatenate([r4, padn], -1).reshape(B * N, _LG), nrow], 0)
    return (tx0, tx1, tx2, tr)


def kernel(X, C, S, edge_idx):
    B, N, A, _ = X.shape
    Kn = edge_idx.shape[2]
    n_edges = B * N * _KP
    rep = _KP * _LG // 128

    tables = _build_tables(X, C, S)

    eidx = edge_idx.astype(jnp.int32)
    gidx = eidx + (jnp.arange(B, dtype=jnp.int32) * N)[:, None, None]
    pvals = (B * N + (jnp.arange(B * N * (_KP - Kn), dtype=jnp.int32) % _NPOIS)
             ).reshape(B, N, _KP - Kn)
    gidx = jnp.concatenate([gidx, pvals], axis=-1)
    nw = 32
    n_chains = 4
    cn = B * N // n_chains
    ne_c = cn * _KP
    gflat = gidx.reshape(B * N, _KP)
    scat = jnp.concatenate([t[:B * N] for t in tables], axis=-1)

    ms_parts = []
    for c in range(n_chains):
        gidx3 = gflat[c * cn:(c + 1) * cn].reshape(nw, ne_c // (nw * _IW), _IW)
        g = _sc_gather4(tables, gidx3, ne_c)
        gxc = tuple(t.reshape(ne_c * _LG // 128, 128) for t in g)
        ssqc = jnp.repeat(scat[c * cn:(c + 1) * cn], rep, axis=0)
        main = _tc_compute(gxc, ssqc, cn * rep)
        ms_parts.append(main.reshape(cn, rep).sum(-1))
    ms = jnp.concatenate(ms_parts).reshape(B, N)

    sx = tuple(t[:B * N].reshape(B, N, _LG) for t in tables)
    ds2 = sum((s[:, :, 4:A, None] - s[:, :, None, :]) ** 2 for s in sx[:3]) + _EPS4
    args = (sx[3][:, :, 4:A, None] + sx[3][:, :, None, :]) - jnp.sqrt(ds2)
    ss = jnp.sum(jnp.tanh(args), axis=(-1, -2))

    scnt = jnp.sum((eidx == jnp.arange(N, dtype=jnp.int32).reshape(1, N, 1))
                   .astype(jnp.float32), axis=-1)
    npair = 10.0 * _KP * _LG
    return 0.5 * (ms + npair) - scnt * (0.5 * (ss + 10.0 * _LG))
